# baseline (speedup 1.0000x reference)
import functools
import math

import jax
import jax.numpy as jnp
from jax import lax
from jax.experimental import pallas as pl
from jax.experimental.pallas import tpu as pltpu
from jax.experimental.pallas import tpu_sc as plsc

N_NODES = 10000
N_PAD = 10112
N_EDGES = 320000
E_PAD = 327680
CHUNK = 128
N_TILES = 16
CHUNKS_PER_TILE = E_PAD // (N_TILES * CHUNK)
WB_BIG = 640
WB_LAST = N_NODES - 15 * WB_BIG
ROWS_PER_TILE_Z = N_PAD // N_TILES
HIDDEN = 256
EMB = 768
G = 128
LOG2 = math.log(2.0)
BLK = 1000
NBLK = N_NODES // BLK


HW = 128


@functools.cache
def _edge_agg():
    mesh = plsc.VectorSubcoreMesh(core_axis_name="c", subcore_axis_name="s")
    out_t = (jax.ShapeDtypeStruct((N_NODES, HW), jnp.float32),
             jax.ShapeDtypeStruct((N_NODES, HW), jnp.float32))
    cpt = CHUNKS_PER_TILE

    @functools.partial(
        pl.kernel, mesh=mesh, out_type=out_t,
        scratch_types=[
            pltpu.VMEM((cpt // 4, CHUNK), jnp.int32),
            pltpu.VMEM((cpt // 4, CHUNK), jnp.int32),
            pltpu.VMEM((CHUNK, HW), jnp.float32),
            pltpu.VMEM((CHUNK, HW), jnp.float32),
            pltpu.VMEM((16,), jnp.int32),
            pltpu.VMEM_SHARED((N_PAD, HW), jnp.float32),
            pltpu.SemaphoreType.DMA,
            pltpu.SemaphoreType.DMA,
            pltpu.SemaphoreType.DMA,
            pltpu.SemaphoreType.DMA,
        ],
    )
    def k(t_a, t_b, src2, dst2, zeros, cfg, out_a, out_b,
          src_v, dst_v, rows_a, rows_b, cfg_v, acc, sg0, sg1, ss0, ss1):
        c = lax.axis_index("c")
        s = lax.axis_index("s")
        pltpu.sync_copy(zeros.at[pl.ds(s * ROWS_PER_TILE_Z, ROWS_PER_TILE_Z)],
                        acc.at[pl.ds(s * ROWS_PER_TILE_Z, ROWS_PER_TILE_Z)])
        pltpu.sync_copy(cfg, cfg_v)
        cfgv = cfg_v[...]
        q = cpt // 4
        n_stages = jnp.where(c == 0, cfgv[0], cfgv[1])
        ebase = jnp.where(c == 0, s * q * cfgv[0],
                          N_TILES * q * cfgv[0] + s * q * cfgv[1])
        plsc.subcore_barrier()

        def run(table):
            def drain_scatters():
                pltpu.make_async_copy(table.at[pl.ds(0, CHUNK)], rows_a,
                                      ss0).wait()
                pltpu.make_async_copy(table.at[pl.ds(0, CHUNK)], rows_b,
                                      ss1).wait()

            def stage_body(st, carry):
                @pl.when(st > 0)
                def _():
                    drain_scatters()
                off = pl.multiple_of(ebase + st * q, 8)
                pltpu.sync_copy(src2.at[pl.ds(off, q)], src_v)
                pltpu.sync_copy(dst2.at[pl.ds(off, q)], dst_v)

                def body(kk, carry2):
                    j0 = 2 * kk
                    j1 = 2 * kk + 1
                    @pl.when(kk > 0)
                    def _():
                        drain_scatters()
                    ga = pltpu.async_copy(table.at[src_v.at[j0]], rows_a, sg0)
                    gb = pltpu.async_copy(table.at[src_v.at[j1]], rows_b, sg1)
                    ga.wait()
                    pltpu.async_copy(rows_a, acc.at[dst_v.at[j0]], ss0,
                                     add=True)
                    gb.wait()
                    pltpu.async_copy(rows_b, acc.at[dst_v.at[j1]], ss1,
                                     add=True)
                    return carry2
                lax.fori_loop(0, q // 2, body, 0)
                return carry
            lax.fori_loop(0, n_stages, stage_body, 0)

            @pl.when(n_stages > 0)
            def _():
                drain_scatters()

        @pl.when(c == 0)
        def _():
            run(t_a)

        @pl.when(c == 1)
        def _():
            run(t_b)

        plsc.subcore_barrier()

        def wb(out):
            @pl.when(s < 15)
            def _():
                pltpu.sync_copy(acc.at[pl.ds(s * WB_BIG, WB_BIG)],
                                out.at[pl.ds(s * WB_BIG, WB_BIG)])

            @pl.when(s == 15)
            def _():
                pltpu.sync_copy(acc.at[pl.ds(15 * WB_BIG, WB_LAST)],
                                out.at[pl.ds(15 * WB_BIG, WB_LAST)])

        @pl.when(c == 0)
        def _():
            wb(out_a)

        @pl.when(c == 1)
        def _():
            wb(out_b)

    return k


def _copy2d(n, m):
    def body(x_ref, o_ref):
        o_ref[...] = x_ref[...]

    return pl.pallas_call(
        body, grid=(NBLK,),
        in_specs=[pl.BlockSpec((n // NBLK, m), lambda i: (i, 0))],
        out_specs=pl.BlockSpec((n // NBLK, m), lambda i: (i, 0)),
        out_shape=jax.ShapeDtypeStruct((n, m), jnp.float32),
    )


def _gin_mlp(d_in, sum_parts):
    n_agg = 2
    aw = d_in if sum_parts else d_in // 2

    def body(*refs):
        h_ref = refs[0]
        agg_refs = refs[1:1 + n_agg]
        wa_ref, ba_ref, wb_ref, bb_ref = refs[1 + n_agg:5 + n_agg]
        t_ref, cs_ref, cq_ref = refs[5 + n_agg:]
        i = pl.program_id(0)
        if sum_parts:
            agg = agg_refs[0][...] + agg_refs[1][...]
        else:
            agg = jnp.concatenate([agg_refs[0][...], agg_refs[1][...]], axis=1)
        z = h_ref[...] + agg
        t1 = jnp.dot(z, wa_ref[...], preferred_element_type=jnp.float32)
        t1 = jnp.maximum(t1 + ba_ref[...], 0.0)
        t2 = jnp.dot(t1, wb_ref[...], preferred_element_type=jnp.float32)
        t2 = jnp.maximum(t2 + bb_ref[...], 0.0)
        t_ref[...] = t2

        @pl.when(i == 0)
        def _():
            cs_ref[...] = jnp.zeros_like(cs_ref)
            cq_ref[...] = jnp.zeros_like(cq_ref)

        cs_ref[...] += jnp.sum(t2, axis=0, keepdims=True)
        cq_ref[...] += jnp.sum(t2 * t2, axis=0, keepdims=True)

    return pl.pallas_call(
        body, grid=(NBLK,),
        in_specs=[
            pl.BlockSpec((BLK, d_in), lambda i: (i, 0)),
        ] + [
            pl.BlockSpec((BLK, aw), lambda i: (i, 0))
            for _ in range(n_agg)
        ] + [
            pl.BlockSpec((d_in, HIDDEN), lambda i: (0, 0)),
            pl.BlockSpec((1, HIDDEN), lambda i: (0, 0)),
            pl.BlockSpec((HIDDEN, HIDDEN), lambda i: (0, 0)),
            pl.BlockSpec((1, HIDDEN), lambda i: (0, 0)),
        ],
        out_specs=[
            pl.BlockSpec((BLK, HIDDEN), lambda i: (i, 0)),
            pl.BlockSpec((1, HIDDEN), lambda i: (0, 0)),
            pl.BlockSpec((1, HIDDEN), lambda i: (0, 0)),
        ],
        out_shape=[
            jax.ShapeDtypeStruct((N_NODES, HIDDEN), jnp.float32),
            jax.ShapeDtypeStruct((1, HIDDEN), jnp.float32),
            jax.ShapeDtypeStruct((1, HIDDEN), jnp.float32),
        ],
    )


def _norm_pool():
    def body(t_ref, cs_ref, cq_ref, b_ref, tn_ref, y_ref):
        i = pl.program_id(0)
        mean = cs_ref[...] * (1.0 / N_NODES)
        var = cq_ref[...] * (1.0 / N_NODES) - mean * mean
        inv = lax.rsqrt(var + 1e-5)
        tn = (t_ref[...] - mean) * inv
        tn_ref[...] = tn
        ids = b_ref[...]
        oh = (ids == lax.broadcasted_iota(jnp.int32, (BLK, G), 1))
        oh = oh.astype(jnp.float32)

        @pl.when(i == 0)
        def _():
            y_ref[...] = jnp.zeros_like(y_ref)

        y_ref[...] += lax.dot_general(oh, tn, (((0,), (0,)), ((), ())),
                                      preferred_element_type=jnp.float32)

    return pl.pallas_call(
        body, grid=(NBLK,),
        in_specs=[
            pl.BlockSpec((BLK, HIDDEN), lambda i: (i, 0)),
            pl.BlockSpec((1, HIDDEN), lambda i: (0, 0)),
            pl.BlockSpec((1, HIDDEN), lambda i: (0, 0)),
            pl.BlockSpec((BLK, 1), lambda i: (i, 0)),
        ],
        out_specs=[
            pl.BlockSpec((BLK, HIDDEN), lambda i: (i, 0)),
            pl.BlockSpec((G, HIDDEN), lambda i: (0, 0)),
        ],
        out_shape=[
            jax.ShapeDtypeStruct((N_NODES, HIDDEN), jnp.float32),
            jax.ShapeDtypeStruct((G, HIDDEN), jnp.float32),
        ],
    )


def _ff_global():
    def body(y_ref, w0, b0, w1, b1, w2, b2, ws, bs, g_ref):
        y = y_ref[...]
        h = jnp.maximum(jnp.dot(y, w0[...], preferred_element_type=jnp.float32) + b0[...], 0.0)
        h = jnp.maximum(jnp.dot(h, w1[...], preferred_element_type=jnp.float32) + b1[...], 0.0)
        h = jnp.maximum(jnp.dot(h, w2[...], preferred_element_type=jnp.float32) + b2[...], 0.0)
        g_ref[...] = h + jnp.dot(y, ws[...], preferred_element_type=jnp.float32) + bs[...]

    return pl.pallas_call(
        body,
        out_shape=jax.ShapeDtypeStruct((G, EMB), jnp.float32),
    )


def _ff_local_loss():
    def body(m_ref, w0, b0, w1, b1, w2, b2, ws, bs, g_ref, b2d_ref,
             loss_ref, acc):
        i = pl.program_id(0)

        @pl.when(i == 0)
        def _():
            acc[0] = 0.0
            acc[1] = 0.0

        m = m_ref[...]
        h = jnp.maximum(jnp.dot(m, w0[...], preferred_element_type=jnp.float32) + b0[...], 0.0)
        h = jnp.maximum(jnp.dot(h, w1[...], preferred_element_type=jnp.float32) + b1[...], 0.0)
        h = jnp.maximum(jnp.dot(h, w2[...], preferred_element_type=jnp.float32) + b2[...], 0.0)
        l = h + jnp.dot(m, ws[...], preferred_element_type=jnp.float32) + bs[...]
        res = lax.dot_general(l, g_ref[...], (((1,), (1,)), ((), ())),
                              preferred_element_type=jnp.float32)
        ids = b2d_ref[...]
        pos = (ids == lax.broadcasted_iota(jnp.int32, (BLK, G), 1))
        pos = pos.astype(jnp.float32)

        def sp(z):
            return jnp.maximum(z, 0.0) + jnp.log1p(jnp.exp(-jnp.abs(z)))

        rp = res * pos
        epos = jnp.sum(LOG2 - sp(-rp))
        q = res * (1.0 - pos)
        eneg = jnp.sum(sp(-q) + q - LOG2)
        acc[0] += epos
        acc[1] += eneg

        @pl.when(i == NBLK - 1)
        def _():
            v = acc[1] / (N_NODES * (G - 1)) - acc[0] / N_NODES
            loss_ref[...] = jnp.reshape(v, (1, 1))

    return pl.pallas_call(
        body, grid=(NBLK,),
        in_specs=[
            pl.BlockSpec((BLK, EMB), lambda i: (i, 0)),
            pl.BlockSpec((EMB, EMB), lambda i: (0, 0)),
            pl.BlockSpec((1, EMB), lambda i: (0, 0)),
            pl.BlockSpec((EMB, EMB), lambda i: (0, 0)),
            pl.BlockSpec((1, EMB), lambda i: (0, 0)),
            pl.BlockSpec((EMB, EMB), lambda i: (0, 0)),
            pl.BlockSpec((1, EMB), lambda i: (0, 0)),
            pl.BlockSpec((EMB, EMB), lambda i: (0, 0)),
            pl.BlockSpec((1, EMB), lambda i: (0, 0)),
            pl.BlockSpec((G, EMB), lambda i: (0, 0)),
            pl.BlockSpec((BLK, 1), lambda i: (i, 0)),
        ],
        out_specs=pl.BlockSpec((1, 1), lambda i: (0, 0)),
        out_shape=jax.ShapeDtypeStruct((1, 1), jnp.float32),
        scratch_shapes=[pltpu.SMEM((2,), jnp.float32)],
    )


def kernel(x, edge_index, batch, num_graphs, params):
    src = edge_index[0]
    dst = edge_index[1]
    pad = E_PAD - N_EDGES
    src1 = jnp.concatenate([src, jnp.zeros((pad,), jnp.int32)])
    dummy_dst = N_NODES + jnp.arange(pad, dtype=jnp.int32) % (N_PAD - N_NODES)
    dst1 = jnp.concatenate([dst, dummy_dst])
    src2 = jnp.concatenate([src1, src1]).reshape(-1, CHUNK)
    dst2 = jnp.concatenate([dst1, dst1]).reshape(-1, CHUNK)
    zeros = jnp.zeros((N_PAD, HW), jnp.float32)
    cfg_half = jnp.zeros((16,), jnp.int32).at[0].set(4).at[1].set(0)
    cfg_full = jnp.zeros((16,), jnp.int32).at[0].set(4).at[1].set(4)
    batch2d = batch.reshape(N_NODES, 1)

    h = x
    xs = []
    ys = []
    for i in range(3):
        d_in = h.shape[1]
        sum_parts = d_in == HW
        if sum_parts:
            a_a, a_b = _edge_agg()(h, _copy2d(N_NODES, HW)(h),
                                   src2, dst2, zeros, cfg_half)
        else:
            a_a, a_b = _edge_agg()(h[:, :HW], h[:, HW:], src2, dst2, zeros,
                                   cfg_full)
        wa = params['gin%d_Wa' % i]
        ba = params['gin%d_ba' % i].reshape(1, HIDDEN)
        wb = params['gin%d_Wb' % i]
        bb = params['gin%d_bb' % i].reshape(1, HIDDEN)
        t_raw, cs, cq = _gin_mlp(d_in, sum_parts)(h, a_a, a_b,
                                                  wa, ba, wb, bb)
        tn, y_i = _norm_pool()(t_raw, cs, cq, batch2d)
        xs.append(tn)
        ys.append(y_i)
        h = tn

    y = jnp.concatenate(ys, axis=1)
    m = jnp.concatenate(xs, axis=1)

    gp = [params['global_W0'], params['global_b0'].reshape(1, EMB),
          params['global_W1'], params['global_b1'].reshape(1, EMB),
          params['global_W2'], params['global_b2'].reshape(1, EMB),
          params['global_Ws'], params['global_bs'].reshape(1, EMB)]
    g_enc = _ff_global()(y, *gp)

    lp = [params['local_W0'], params['local_b0'].reshape(1, EMB),
          params['local_W1'], params['local_b1'].reshape(1, EMB),
          params['local_W2'], params['local_b2'].reshape(1, EMB),
          params['local_Ws'], params['local_bs'].reshape(1, EMB)]
    loss = _ff_local_loss()(m, *lp, g_enc, batch2d)
    return loss[0, 0]


# speedup vs baseline: 1.1211x; 1.1211x over previous
import functools
import math

import jax
import jax.numpy as jnp
from jax import lax
from jax.experimental import pallas as pl
from jax.experimental.pallas import tpu as pltpu
from jax.experimental.pallas import tpu_sc as plsc

N_NODES = 10000
N_PAD = 10112
N_EDGES = 320000
E_PAD = 327680
CHUNK = 128
N_TILES = 16
CHUNKS_PER_TILE = E_PAD // (N_TILES * CHUNK)
WB_BIG = 640
WB_LAST = N_NODES - 15 * WB_BIG
ROWS_PER_TILE_Z = N_PAD // N_TILES
HIDDEN = 256
EMB = 768
G = 128
LOG2 = math.log(2.0)
BLK = 1000
NBLK = N_NODES // BLK


HW = 128


@functools.cache
def _edge_agg():
    mesh = plsc.VectorSubcoreMesh(core_axis_name="c", subcore_axis_name="s")
    out_t = (jax.ShapeDtypeStruct((N_NODES, HW), jnp.float32),
             jax.ShapeDtypeStruct((N_NODES, HW), jnp.float32))
    cpt = CHUNKS_PER_TILE

    @functools.partial(
        pl.kernel, mesh=mesh, out_type=out_t,
        scratch_types=[
            pltpu.VMEM((cpt // 4, CHUNK), jnp.int32),
            pltpu.VMEM((cpt // 4, CHUNK), jnp.int32),
            pltpu.VMEM((CHUNK, HW), jnp.float32),
            pltpu.VMEM((CHUNK, HW), jnp.float32),
            pltpu.VMEM((16,), jnp.int32),
            pltpu.VMEM_SHARED((N_PAD, HW), jnp.float32),
            pltpu.SemaphoreType.DMA,
            pltpu.SemaphoreType.DMA,
            pltpu.SemaphoreType.DMA,
            pltpu.SemaphoreType.DMA,
        ],
    )
    def k(t_a, t_b, src2, dst2, zeros, cfg, out_a, out_b,
          src_v, dst_v, rows_a, rows_b, cfg_v, acc, sg0, sg1, ss0, ss1):
        c = lax.axis_index("c")
        s = lax.axis_index("s")
        pltpu.sync_copy(zeros.at[pl.ds(s * ROWS_PER_TILE_Z, ROWS_PER_TILE_Z)],
                        acc.at[pl.ds(s * ROWS_PER_TILE_Z, ROWS_PER_TILE_Z)])
        pltpu.sync_copy(cfg, cfg_v)
        cfgv = cfg_v[...]
        q = cpt // 4
        n_stages = jnp.where(c == 0, cfgv[0], cfgv[1])
        ebase = jnp.where(c == 0, s * q * cfgv[0],
                          N_TILES * q * cfgv[0] + s * q * cfgv[1])
        plsc.subcore_barrier()

        def run(table):
            def drain_scatters():
                pltpu.make_async_copy(table.at[pl.ds(0, CHUNK)], rows_a,
                                      ss0).wait()
                pltpu.make_async_copy(table.at[pl.ds(0, CHUNK)], rows_b,
                                      ss1).wait()

            def stage_body(st, carry):
                @pl.when(st > 0)
                def _():
                    drain_scatters()
                off = pl.multiple_of(ebase + st * q, 8)
                pltpu.sync_copy(src2.at[pl.ds(off, q)], src_v)
                pltpu.sync_copy(dst2.at[pl.ds(off, q)], dst_v)

                def body(kk, carry2):
                    j0 = 2 * kk
                    j1 = 2 * kk + 1
                    @pl.when(kk > 0)
                    def _():
                        drain_scatters()
                    ga = pltpu.async_copy(table.at[src_v.at[j0]], rows_a, sg0)
                    gb = pltpu.async_copy(table.at[src_v.at[j1]], rows_b, sg1)
                    ga.wait()
                    pltpu.async_copy(rows_a, acc.at[dst_v.at[j0]], ss0,
                                     add=True)
                    gb.wait()
                    pltpu.async_copy(rows_b, acc.at[dst_v.at[j1]], ss1,
                                     add=True)
                    return carry2
                lax.fori_loop(0, q // 2, body, 0)
                return carry
            lax.fori_loop(0, n_stages, stage_body, 0)

            @pl.when(n_stages > 0)
            def _():
                drain_scatters()

        @pl.when(c == 0)
        def _():
            run(t_a)

        @pl.when(c == 1)
        def _():
            run(t_b)

        plsc.subcore_barrier()

        def wb(out):
            @pl.when(s < 15)
            def _():
                pltpu.sync_copy(acc.at[pl.ds(s * WB_BIG, WB_BIG)],
                                out.at[pl.ds(s * WB_BIG, WB_BIG)])

            @pl.when(s == 15)
            def _():
                pltpu.sync_copy(acc.at[pl.ds(15 * WB_BIG, WB_LAST)],
                                out.at[pl.ds(15 * WB_BIG, WB_LAST)])

        @pl.when(c == 0)
        def _():
            wb(out_a)

        @pl.when(c == 1)
        def _():
            wb(out_b)

    return k


def _copy2d(n, m):
    def body(x_ref, o_ref):
        o_ref[...] = x_ref[...]

    return pl.pallas_call(
        body, grid=(NBLK,),
        in_specs=[pl.BlockSpec((n // NBLK, m), lambda i: (i, 0))],
        out_specs=pl.BlockSpec((n // NBLK, m), lambda i: (i, 0)),
        out_shape=jax.ShapeDtypeStruct((n, m), jnp.float32),
    )


def _gin_mlp(d_in, sum_parts):
    n_agg = 2
    aw = d_in if sum_parts else d_in // 2

    def body(*refs):
        h_ref = refs[0]
        agg_refs = refs[1:1 + n_agg]
        wa_ref, ba_ref, wb_ref, bb_ref = refs[1 + n_agg:5 + n_agg]
        t_ref, cs_ref, cq_ref = refs[5 + n_agg:]
        i = pl.program_id(0)
        if sum_parts:
            agg = agg_refs[0][...] + agg_refs[1][...]
        else:
            agg = jnp.concatenate([agg_refs[0][...], agg_refs[1][...]], axis=1)
        z = h_ref[...] + agg
        t1 = jnp.dot(z, wa_ref[...], preferred_element_type=jnp.float32)
        t1 = jnp.maximum(t1 + ba_ref[...], 0.0)
        t2 = jnp.dot(t1, wb_ref[...], preferred_element_type=jnp.float32)
        t2 = jnp.maximum(t2 + bb_ref[...], 0.0)
        t_ref[...] = t2

        @pl.when(i == 0)
        def _():
            cs_ref[...] = jnp.zeros_like(cs_ref)
            cq_ref[...] = jnp.zeros_like(cq_ref)

        cs_ref[...] += jnp.sum(t2, axis=0, keepdims=True)
        cq_ref[...] += jnp.sum(t2 * t2, axis=0, keepdims=True)

    return pl.pallas_call(
        body, grid=(NBLK,),
        in_specs=[
            pl.BlockSpec((BLK, d_in), lambda i: (i, 0)),
        ] + [
            pl.BlockSpec((BLK, aw), lambda i: (i, 0))
            for _ in range(n_agg)
        ] + [
            pl.BlockSpec((d_in, HIDDEN), lambda i: (0, 0)),
            pl.BlockSpec((1, HIDDEN), lambda i: (0, 0)),
            pl.BlockSpec((HIDDEN, HIDDEN), lambda i: (0, 0)),
            pl.BlockSpec((1, HIDDEN), lambda i: (0, 0)),
        ],
        out_specs=[
            pl.BlockSpec((BLK, HIDDEN), lambda i: (i, 0)),
            pl.BlockSpec((1, HIDDEN), lambda i: (0, 0)),
            pl.BlockSpec((1, HIDDEN), lambda i: (0, 0)),
        ],
        out_shape=[
            jax.ShapeDtypeStruct((N_NODES, HIDDEN), jnp.float32),
            jax.ShapeDtypeStruct((1, HIDDEN), jnp.float32),
            jax.ShapeDtypeStruct((1, HIDDEN), jnp.float32),
        ],
    )


def _norm_pool():
    def body(t_ref, cs_ref, cq_ref, b_ref, tn_ref, lo_ref, hi_ref, y_ref):
        i = pl.program_id(0)
        mean = cs_ref[...] * (1.0 / N_NODES)
        var = cq_ref[...] * (1.0 / N_NODES) - mean * mean
        inv = lax.rsqrt(var + 1e-5)
        tn = (t_ref[...] - mean) * inv
        tn_ref[...] = tn
        lo_ref[...] = tn[:, :HW]
        hi_ref[...] = tn[:, HW:]
        ids = b_ref[...]
        oh = (ids == lax.broadcasted_iota(jnp.int32, (BLK, G), 1))
        oh = oh.astype(jnp.float32)

        @pl.when(i == 0)
        def _():
            y_ref[...] = jnp.zeros_like(y_ref)

        y_ref[...] += lax.dot_general(oh, tn, (((0,), (0,)), ((), ())),
                                      preferred_element_type=jnp.float32)

    return pl.pallas_call(
        body, grid=(NBLK,),
        in_specs=[
            pl.BlockSpec((BLK, HIDDEN), lambda i: (i, 0)),
            pl.BlockSpec((1, HIDDEN), lambda i: (0, 0)),
            pl.BlockSpec((1, HIDDEN), lambda i: (0, 0)),
            pl.BlockSpec((BLK, 1), lambda i: (i, 0)),
        ],
        out_specs=[
            pl.BlockSpec((BLK, HIDDEN), lambda i: (i, 0)),
            pl.BlockSpec((BLK, HW), lambda i: (i, 0)),
            pl.BlockSpec((BLK, HW), lambda i: (i, 0)),
            pl.BlockSpec((G, HIDDEN), lambda i: (0, 0)),
        ],
        out_shape=[
            jax.ShapeDtypeStruct((N_NODES, HIDDEN), jnp.float32),
            jax.ShapeDtypeStruct((N_NODES, HW), jnp.float32),
            jax.ShapeDtypeStruct((N_NODES, HW), jnp.float32),
            jax.ShapeDtypeStruct((G, HIDDEN), jnp.float32),
        ],
    )


def _ff_global():
    def body(y_ref, w0, b0, w1, b1, w2, b2, ws, bs, g_ref):
        y = y_ref[...]
        h = jnp.maximum(jnp.dot(y, w0[...], preferred_element_type=jnp.float32) + b0[...], 0.0)
        h = jnp.maximum(jnp.dot(h, w1[...], preferred_element_type=jnp.float32) + b1[...], 0.0)
        h = jnp.maximum(jnp.dot(h, w2[...], preferred_element_type=jnp.float32) + b2[...], 0.0)
        g_ref[...] = h + jnp.dot(y, ws[...], preferred_element_type=jnp.float32) + bs[...]

    return pl.pallas_call(
        body,
        out_shape=jax.ShapeDtypeStruct((G, EMB), jnp.float32),
    )


def _ff_local_loss():
    def body(m_ref, w0, b0, w1, b1, w2, b2, ws, bs, g_ref, b2d_ref,
             loss_ref, acc):
        i = pl.program_id(0)

        @pl.when(i == 0)
        def _():
            acc[0] = 0.0
            acc[1] = 0.0

        m = m_ref[...]
        h = jnp.maximum(jnp.dot(m, w0[...], preferred_element_type=jnp.float32) + b0[...], 0.0)
        h = jnp.maximum(jnp.dot(h, w1[...], preferred_element_type=jnp.float32) + b1[...], 0.0)
        h = jnp.maximum(jnp.dot(h, w2[...], preferred_element_type=jnp.float32) + b2[...], 0.0)
        l = h + jnp.dot(m, ws[...], preferred_element_type=jnp.float32) + bs[...]
        res = lax.dot_general(l, g_ref[...], (((1,), (1,)), ((), ())),
                              preferred_element_type=jnp.float32)
        ids = b2d_ref[...]
        pos = (ids == lax.broadcasted_iota(jnp.int32, (BLK, G), 1))
        pos = pos.astype(jnp.float32)

        def sp(z):
            return jnp.maximum(z, 0.0) + jnp.log1p(jnp.exp(-jnp.abs(z)))

        rp = res * pos
        epos = jnp.sum(LOG2 - sp(-rp))
        q = res * (1.0 - pos)
        eneg = jnp.sum(sp(-q) + q - LOG2)
        acc[0] += epos
        acc[1] += eneg

        @pl.when(i == NBLK - 1)
        def _():
            v = acc[1] / (N_NODES * (G - 1)) - acc[0] / N_NODES
            loss_ref[...] = jnp.reshape(v, (1, 1))

    return pl.pallas_call(
        body, grid=(NBLK,),
        in_specs=[
            pl.BlockSpec((BLK, EMB), lambda i: (i, 0)),
            pl.BlockSpec((EMB, EMB), lambda i: (0, 0)),
            pl.BlockSpec((1, EMB), lambda i: (0, 0)),
            pl.BlockSpec((EMB, EMB), lambda i: (0, 0)),
            pl.BlockSpec((1, EMB), lambda i: (0, 0)),
            pl.BlockSpec((EMB, EMB), lambda i: (0, 0)),
            pl.BlockSpec((1, EMB), lambda i: (0, 0)),
            pl.BlockSpec((EMB, EMB), lambda i: (0, 0)),
            pl.BlockSpec((1, EMB), lambda i: (0, 0)),
            pl.BlockSpec((G, EMB), lambda i: (0, 0)),
            pl.BlockSpec((BLK, 1), lambda i: (i, 0)),
        ],
        out_specs=pl.BlockSpec((1, 1), lambda i: (0, 0)),
        out_shape=jax.ShapeDtypeStruct((1, 1), jnp.float32),
        scratch_shapes=[pltpu.SMEM((2,), jnp.float32)],
    )


def kernel(x, edge_index, batch, num_graphs, params):
    src = edge_index[0]
    dst = edge_index[1]
    pad = E_PAD - N_EDGES
    src1 = jnp.concatenate([src, jnp.zeros((pad,), jnp.int32)])
    dummy_dst = N_NODES + jnp.arange(pad, dtype=jnp.int32) % (N_PAD - N_NODES)
    dst1 = jnp.concatenate([dst, dummy_dst])
    src2 = jnp.concatenate([src1, src1]).reshape(-1, CHUNK)
    dst2 = jnp.concatenate([dst1, dst1]).reshape(-1, CHUNK)
    zeros = jnp.zeros((N_PAD, HW), jnp.float32)
    cfg_half = jnp.zeros((16,), jnp.int32).at[0].set(3).at[1].set(1)
    cfg_full = jnp.zeros((16,), jnp.int32).at[0].set(4).at[1].set(4)
    batch2d = batch.reshape(N_NODES, 1)

    h = x
    h_halves = None
    xs = []
    ys = []
    for i in range(3):
        d_in = h.shape[1]
        sum_parts = d_in == HW
        if sum_parts:
            a_a, a_b = _edge_agg()(h, _copy2d(N_NODES, HW)(h),
                                   src2, dst2, zeros, cfg_half)
        else:
            a_a, a_b = _edge_agg()(h_halves[0], h_halves[1],
                                   src2, dst2, zeros, cfg_full)
        wa = params['gin%d_Wa' % i]
        ba = params['gin%d_ba' % i].reshape(1, HIDDEN)
        wb = params['gin%d_Wb' % i]
        bb = params['gin%d_bb' % i].reshape(1, HIDDEN)
        t_raw, cs, cq = _gin_mlp(d_in, sum_parts)(h, a_a, a_b,
                                                  wa, ba, wb, bb)
        tn, tn_lo, tn_hi, y_i = _norm_pool()(t_raw, cs, cq, batch2d)
        xs.append(tn)
        ys.append(y_i)
        h = tn
        h_halves = (tn_lo, tn_hi)

    y = jnp.concatenate(ys, axis=1)
    m = jnp.concatenate(xs, axis=1)

    gp = [params['global_W0'], params['global_b0'].reshape(1, EMB),
          params['global_W1'], params['global_b1'].reshape(1, EMB),
          params['global_W2'], params['global_b2'].reshape(1, EMB),
          params['global_Ws'], params['global_bs'].reshape(1, EMB)]
    g_enc = _ff_global()(y, *gp)

    lp = [params['local_W0'], params['local_b0'].reshape(1, EMB),
          params['local_W1'], params['local_b1'].reshape(1, EMB),
          params['local_W2'], params['local_b2'].reshape(1, EMB),
          params['local_Ws'], params['local_bs'].reshape(1, EMB)]
    loss = _ff_local_loss()(m, *lp, g_enc, batch2d)
    return loss[0, 0]
